# baseline (speedup 1.0000x reference)
import functools

import jax
import jax.numpy as jnp
from jax import lax
from jax.experimental import pallas as pl
from jax.experimental.pallas import tpu as pltpu
from jax.experimental.pallas import tpu_sc as plsc

NUM_EMB_K = 100000
EMB_D = 64
HASHED_K = 64000 * 10
BATCH_K = 4096
BAG_K = 50

NC = 2
NS = 16
NW = NC * NS

A_ROWS_PER_TILE = NUM_EMB_K // NW
A_CHUNK_ROWS = 125
A_CHUNKS = A_ROWS_PER_TILE // A_CHUNK_ROWS
A_CHUNK_ELEMS = A_CHUNK_ROWS * EMB_D
A_HBM_PART = 0
A_SP_PART = A_CHUNK_ELEMS - A_HBM_PART

B_BAGS_PER_TILE = BATCH_K // NW
B_GROUP = 16
B_GROUPS = B_BAGS_PER_TILE // B_GROUP
B_GROUP_IDS = B_GROUP * BAG_K


def _wid():
    return lax.axis_index("s") * NC + lax.axis_index("c")


HW_SHARD = HASHED_K // NS
CH = A_CHUNK_ELEMS

A_SCRATCH = [
    pltpu.VMEM((2 * CH,), jnp.int32),
    pltpu.VMEM((2 * CH,), jnp.int32),
    pltpu.VMEM((2 * CH,), jnp.float32),
    pltpu.VMEM((2 * CH,), jnp.float32),
    pltpu.VMEM((2 * CH,), jnp.float32),
    pltpu.VMEM_SHARED((HASHED_K,), jnp.float32),
    pltpu.SemaphoreType.DMA,
    pltpu.SemaphoreType.DMA,
    pltpu.SemaphoreType.DMA,
    pltpu.SemaphoreType.DMA,
    pltpu.SemaphoreType.DMA,
    pltpu.SemaphoreType.DMA,
    pltpu.SemaphoreType.DMA,
    pltpu.SemaphoreType.DMA,
]


def _build_table_body(
    hw, idx0f, idx1f, tflat,
    i0d, i1d, v0d, v1d, td, hw_sh,
    gh0, gh1, gs0, gs1, ix0, ix1, wr0, wr1,
):
    sid = lax.axis_index("s")
    sem_gh = (gh0, gh1)
    sem_gs = (gs0, gs1)
    sem_ix = (ix0, ix1)
    sem_wr = (wr0, wr1)

    def stage(k, _):
        off = sid * HW_SHARD + k * CH
        pltpu.sync_copy(hw.at[pl.ds(off, CH)], v0d.at[pl.ds(0, CH)])
        pltpu.sync_copy(v0d.at[pl.ds(0, CH)], hw_sh.at[pl.ds(off, CH)])
        return 0

    lax.fori_loop(0, HW_SHARD // CH, stage, 0)
    plsc.subcore_barrier()

    base = _wid() * A_ROWS_PER_TILE * EMB_D

    def d_idx(cj, b):
        off = base + cj * CH
        return (
            pltpu.make_async_copy(
                idx0f.at[pl.ds(off, CH)], i0d.at[pl.ds(b * CH, CH)], sem_ix[b]
            ),
            pltpu.make_async_copy(
                idx1f.at[pl.ds(off, CH)], i1d.at[pl.ds(b * CH, CH)], sem_ix[b]
            ),
        )

    def d_gather(b):
        o = b * CH
        hbm_part = (
            (
                pltpu.make_async_copy(
                    hw.at[i0d.at[pl.ds(o, A_HBM_PART)]],
                    v0d.at[pl.ds(o, A_HBM_PART)],
                    sem_gh[b],
                ),
                pltpu.make_async_copy(
                    hw.at[i1d.at[pl.ds(o, A_HBM_PART)]],
                    v1d.at[pl.ds(o, A_HBM_PART)],
                    sem_gh[b],
                ),
            )
            if A_HBM_PART
            else ()
        )
        return hbm_part + (
            pltpu.make_async_copy(
                hw_sh.at[i0d.at[pl.ds(o + A_HBM_PART, A_SP_PART)]],
                v0d.at[pl.ds(o + A_HBM_PART, A_SP_PART)],
                sem_gs[b],
            ),
            pltpu.make_async_copy(
                hw_sh.at[i1d.at[pl.ds(o + A_HBM_PART, A_SP_PART)]],
                v1d.at[pl.ds(o + A_HBM_PART, A_SP_PART)],
                sem_gs[b],
            ),
        )

    def d_wr(cj, b):
        return pltpu.make_async_copy(
            td.at[pl.ds(b * CH, CH)],
            tflat.at[pl.ds(base + cj * CH, CH)],
            sem_wr[b],
        )

    for d in d_idx(0, 0):
        d.start()
    for d in d_idx(0, 0):
        d.wait()
    for d in d_gather(0):
        d.start()
    for d in d_idx(1, 1):
        d.start()

    def half_step(h, _):
        for b in range(2):
            ci = 2 * h + b
            nb = 1 - b

            @pl.when(ci < A_CHUNKS)
            def _():
                for d in d_gather(b):
                    d.wait()

                @pl.when(ci + 1 < A_CHUNKS)
                def _():
                    for d in d_idx(ci + 1, nb):
                        d.wait()
                    for d in d_gather(nb):
                        d.start()

                @pl.when(ci + 2 < A_CHUNKS)
                def _():
                    for d in d_idx(ci + 2, b):
                        d.start()

                @pl.when(ci >= 2)
                def _():
                    d_wr(ci - 2, b).wait()

                def add_vec(k, _):
                    o = b * CH + k * EMB_D
                    for u in range(4):
                        s = pl.ds(o + u * 16, 16)
                        td[s] = v0d[s] + v1d[s]
                    return 0

                lax.fori_loop(0, A_CHUNK_ROWS, add_vec, 0)
                d_wr(ci, b).start()

        return 0

    lax.fori_loop(0, (A_CHUNKS + 1) // 2, half_step, 0)
    d_wr(A_CHUNKS - 2, (A_CHUNKS - 2) % 2).wait()
    d_wr(A_CHUNKS - 1, (A_CHUNKS - 1) % 2).wait()


B_SCRATCH = [
    pltpu.VMEM((2 * B_GROUP_IDS,), jnp.int32),
    pltpu.VMEM((2 * B_GROUP_IDS, EMB_D), jnp.float32),
    pltpu.VMEM((2 * B_GROUP, EMB_D), jnp.float32),
    pltpu.SemaphoreType.DMA,
    pltpu.SemaphoreType.DMA,
    pltpu.SemaphoreType.DMA,
    pltpu.SemaphoreType.DMA,
    pltpu.SemaphoreType.DMA,
    pltpu.SemaphoreType.DMA,
]


def _bag_pool_body(
    table, xflat, out, xg_v, rows_v, out_v, gr0, gr1, ix0, ix1, wr0, wr1
):
    tile_bag0 = _wid() * B_BAGS_PER_TILE
    sem_gr = (gr0, gr1)
    sem_ix = (ix0, ix1)
    sem_wr = (wr0, wr1)

    def d_ids(ci, b):
        bag0 = tile_bag0 + ci * B_GROUP
        return pltpu.make_async_copy(
            xflat.at[pl.ds(bag0 * BAG_K, B_GROUP_IDS)],
            xg_v.at[pl.ds(b * B_GROUP_IDS, B_GROUP_IDS)],
            sem_ix[b],
        )

    def d_rows(b):
        return pltpu.make_async_copy(
            table.at[xg_v.at[pl.ds(b * B_GROUP_IDS, B_GROUP_IDS)]],
            rows_v.at[pl.ds(b * B_GROUP_IDS, B_GROUP_IDS)],
            sem_gr[b],
        )

    def d_out(ci, b):
        bag0 = tile_bag0 + ci * B_GROUP
        return pltpu.make_async_copy(
            out_v.at[pl.ds(b * B_GROUP, B_GROUP)],
            out.at[pl.ds(bag0, B_GROUP)],
            sem_wr[b],
        )

    d_ids(0, 0).start()
    d_ids(0, 0).wait()
    d_rows(0).start()
    d_ids(1, 1).start()

    def half_step(h, _):
        for b in range(2):
            ci = 2 * h + b
            nb = 1 - b

            d_rows(b).wait()

            @pl.when(ci + 1 < B_GROUPS)
            def _():
                d_ids(ci + 1, nb).wait()
                d_rows(nb).start()

            @pl.when(ci + 2 < B_GROUPS)
            def _():
                d_ids(ci + 2, b).start()

            @pl.when(ci >= 2)
            def _():
                d_out(ci - 2, b).wait()

            def bag(g, _):
                def lstep(l, acc):
                    r = b * B_GROUP_IDS + g * BAG_K + l
                    return tuple(
                        acc[c] + rows_v[r, pl.ds(c * 16, 16)] for c in range(4)
                    )

                z = jnp.zeros((16,), jnp.float32)
                acc = lax.fori_loop(0, BAG_K, lstep, (z, z, z, z))
                for c in range(4):
                    out_v[b * B_GROUP + g, pl.ds(c * 16, 16)] = acc[c]
                return 0

            lax.fori_loop(0, B_GROUP, bag, 0)
            d_out(ci, b).start()

        return 0

    lax.fori_loop(0, B_GROUPS // 2, half_step, 0)
    d_out(B_GROUPS - 2, 0).wait()
    d_out(B_GROUPS - 1, 1).wait()


_build_table = pl.kernel(
    _build_table_body,
    out_type=jax.ShapeDtypeStruct((NUM_EMB_K * EMB_D,), jnp.float32),
    mesh=plsc.VectorSubcoreMesh(core_axis_name="c", subcore_axis_name="s", num_cores=NC, num_subcores=NS),
    scratch_types=A_SCRATCH,
    compiler_params=pltpu.CompilerParams(use_tc_tiling_on_sc=False),
)

_bag_pool = pl.kernel(
    _bag_pool_body,
    out_type=jax.ShapeDtypeStruct((BATCH_K, EMB_D), jnp.float32),
    mesh=plsc.VectorSubcoreMesh(core_axis_name="c", subcore_axis_name="s", num_cores=NC, num_subcores=NS),
    scratch_types=B_SCRATCH,
    compiler_params=pltpu.CompilerParams(use_tc_tiling_on_sc=False),
)


def kernel(x, hashed_weight, idx0, idx1):
    xflat = x.reshape(-1)
    i0 = idx0.reshape(-1)
    i1 = idx1.reshape(-1)
    tflat = _build_table(hashed_weight, i0, i1)
    table = tflat.reshape(NUM_EMB_K, EMB_D)
    return _bag_pool(table, xflat)


# speedup vs baseline: 1.0416x; 1.0416x over previous
import functools

import jax
import jax.numpy as jnp
from jax import lax
from jax.experimental import pallas as pl
from jax.experimental.pallas import tpu as pltpu
from jax.experimental.pallas import tpu_sc as plsc

NUM_EMB_K = 100000
EMB_D = 64
HASHED_K = 64000 * 10
BATCH_K = 4096
BAG_K = 50

NC = 2
NS = 16
NW = NC * NS

A_ROWS_PER_TILE = NUM_EMB_K // NW
A_CHUNK_ROWS = 125
A_CHUNKS = A_ROWS_PER_TILE // A_CHUNK_ROWS
A_CHUNK_ELEMS = A_CHUNK_ROWS * EMB_D
A_HBM_PART = 800
A_SP_PART = A_CHUNK_ELEMS - A_HBM_PART

B_BAGS_PER_TILE = BATCH_K // NW
B_GROUP = 16
B_GROUPS = B_BAGS_PER_TILE // B_GROUP
B_GROUP_IDS = B_GROUP * BAG_K


def _wid():
    return lax.axis_index("s") * NC + lax.axis_index("c")


HW_SHARD = HASHED_K // NS
CH = A_CHUNK_ELEMS

A_SCRATCH = [
    pltpu.VMEM((2 * CH,), jnp.int32),
    pltpu.VMEM((2 * CH,), jnp.int32),
    pltpu.VMEM((2 * CH,), jnp.float32),
    pltpu.VMEM((2 * CH,), jnp.float32),
    pltpu.VMEM((2 * CH,), jnp.float32),
    pltpu.VMEM_SHARED((HASHED_K,), jnp.float32),
    pltpu.SemaphoreType.DMA,
    pltpu.SemaphoreType.DMA,
    pltpu.SemaphoreType.DMA,
    pltpu.SemaphoreType.DMA,
    pltpu.SemaphoreType.DMA,
    pltpu.SemaphoreType.DMA,
    pltpu.SemaphoreType.DMA,
    pltpu.SemaphoreType.DMA,
]


def _build_table_body(
    hw, idx0f, idx1f, tflat,
    i0d, i1d, v0d, v1d, td, hw_sh,
    gh0, gh1, gs0, gs1, ix0, ix1, wr0, wr1,
):
    sid = lax.axis_index("s")
    sem_gh = (gh0, gh1)
    sem_gs = (gs0, gs1)
    sem_ix = (ix0, ix1)
    sem_wr = (wr0, wr1)

    def stage(k, _):
        off = sid * HW_SHARD + k * CH
        pltpu.sync_copy(hw.at[pl.ds(off, CH)], v0d.at[pl.ds(0, CH)])
        pltpu.sync_copy(v0d.at[pl.ds(0, CH)], hw_sh.at[pl.ds(off, CH)])
        return 0

    lax.fori_loop(0, HW_SHARD // CH, stage, 0)
    plsc.subcore_barrier()

    base = _wid() * A_ROWS_PER_TILE * EMB_D

    def d_idx(cj, b):
        off = base + cj * CH
        return (
            pltpu.make_async_copy(
                idx0f.at[pl.ds(off, CH)], i0d.at[pl.ds(b * CH, CH)], sem_ix[b]
            ),
            pltpu.make_async_copy(
                idx1f.at[pl.ds(off, CH)], i1d.at[pl.ds(b * CH, CH)], sem_ix[b]
            ),
        )

    def d_gather(b):
        o = b * CH
        hbm_part = (
            (
                pltpu.make_async_copy(
                    hw.at[i0d.at[pl.ds(o, A_HBM_PART)]],
                    v0d.at[pl.ds(o, A_HBM_PART)],
                    sem_gh[b],
                ),
                pltpu.make_async_copy(
                    hw.at[i1d.at[pl.ds(o, A_HBM_PART)]],
                    v1d.at[pl.ds(o, A_HBM_PART)],
                    sem_gh[b],
                ),
            )
            if A_HBM_PART
            else ()
        )
        return hbm_part + (
            pltpu.make_async_copy(
                hw_sh.at[i0d.at[pl.ds(o + A_HBM_PART, A_SP_PART)]],
                v0d.at[pl.ds(o + A_HBM_PART, A_SP_PART)],
                sem_gs[b],
            ),
            pltpu.make_async_copy(
                hw_sh.at[i1d.at[pl.ds(o + A_HBM_PART, A_SP_PART)]],
                v1d.at[pl.ds(o + A_HBM_PART, A_SP_PART)],
                sem_gs[b],
            ),
        )

    def d_wr(cj, b):
        return pltpu.make_async_copy(
            td.at[pl.ds(b * CH, CH)],
            tflat.at[pl.ds(base + cj * CH, CH)],
            sem_wr[b],
        )

    for d in d_idx(0, 0):
        d.start()
    for d in d_idx(0, 0):
        d.wait()
    for d in d_gather(0):
        d.start()
    for d in d_idx(1, 1):
        d.start()

    def half_step(h, _):
        for b in range(2):
            ci = 2 * h + b
            nb = 1 - b

            @pl.when(ci < A_CHUNKS)
            def _():
                for d in d_gather(b):
                    d.wait()

                @pl.when(ci + 1 < A_CHUNKS)
                def _():
                    for d in d_idx(ci + 1, nb):
                        d.wait()
                    for d in d_gather(nb):
                        d.start()

                @pl.when(ci + 2 < A_CHUNKS)
                def _():
                    for d in d_idx(ci + 2, b):
                        d.start()

                @pl.when(ci >= 2)
                def _():
                    d_wr(ci - 2, b).wait()

                def add_vec(k, _):
                    o = b * CH + k * EMB_D
                    for u in range(4):
                        s = pl.ds(o + u * 16, 16)
                        td[s] = v0d[s] + v1d[s]
                    return 0

                lax.fori_loop(0, A_CHUNK_ROWS, add_vec, 0)
                d_wr(ci, b).start()

        return 0

    lax.fori_loop(0, (A_CHUNKS + 1) // 2, half_step, 0)
    d_wr(A_CHUNKS - 2, (A_CHUNKS - 2) % 2).wait()
    d_wr(A_CHUNKS - 1, (A_CHUNKS - 1) % 2).wait()


B_SCRATCH = [
    pltpu.VMEM((2 * B_GROUP_IDS,), jnp.int32),
    pltpu.VMEM((2 * B_GROUP_IDS, EMB_D), jnp.float32),
    pltpu.VMEM((2 * B_GROUP, EMB_D), jnp.float32),
    pltpu.SemaphoreType.DMA,
    pltpu.SemaphoreType.DMA,
    pltpu.SemaphoreType.DMA,
    pltpu.SemaphoreType.DMA,
    pltpu.SemaphoreType.DMA,
    pltpu.SemaphoreType.DMA,
]


def _bag_pool_body(
    table, xflat, out, xg_v, rows_v, out_v, gr0, gr1, ix0, ix1, wr0, wr1
):
    tile_bag0 = _wid() * B_BAGS_PER_TILE
    sem_gr = (gr0, gr1)
    sem_ix = (ix0, ix1)
    sem_wr = (wr0, wr1)

    def d_ids(ci, b):
        bag0 = tile_bag0 + ci * B_GROUP
        return pltpu.make_async_copy(
            xflat.at[pl.ds(bag0 * BAG_K, B_GROUP_IDS)],
            xg_v.at[pl.ds(b * B_GROUP_IDS, B_GROUP_IDS)],
            sem_ix[b],
        )

    def d_rows(b):
        return pltpu.make_async_copy(
            table.at[xg_v.at[pl.ds(b * B_GROUP_IDS, B_GROUP_IDS)]],
            rows_v.at[pl.ds(b * B_GROUP_IDS, B_GROUP_IDS)],
            sem_gr[b],
        )

    def d_out(ci, b):
        bag0 = tile_bag0 + ci * B_GROUP
        return pltpu.make_async_copy(
            out_v.at[pl.ds(b * B_GROUP, B_GROUP)],
            out.at[pl.ds(bag0, B_GROUP)],
            sem_wr[b],
        )

    d_ids(0, 0).start()
    d_ids(0, 0).wait()
    d_rows(0).start()
    d_ids(1, 1).start()

    def half_step(h, _):
        for b in range(2):
            ci = 2 * h + b
            nb = 1 - b

            d_rows(b).wait()

            @pl.when(ci + 1 < B_GROUPS)
            def _():
                d_ids(ci + 1, nb).wait()
                d_rows(nb).start()

            @pl.when(ci + 2 < B_GROUPS)
            def _():
                d_ids(ci + 2, b).start()

            @pl.when(ci >= 2)
            def _():
                d_out(ci - 2, b).wait()

            def bag(g, _):
                def lstep(l, acc):
                    r = b * B_GROUP_IDS + g * BAG_K + l
                    return tuple(
                        acc[c] + rows_v[r, pl.ds(c * 16, 16)] for c in range(4)
                    )

                z = jnp.zeros((16,), jnp.float32)
                acc = lax.fori_loop(0, BAG_K, lstep, (z, z, z, z))
                for c in range(4):
                    out_v[b * B_GROUP + g, pl.ds(c * 16, 16)] = acc[c]
                return 0

            lax.fori_loop(0, B_GROUP, bag, 0)
            d_out(ci, b).start()

        return 0

    lax.fori_loop(0, B_GROUPS // 2, half_step, 0)
    d_out(B_GROUPS - 2, 0).wait()
    d_out(B_GROUPS - 1, 1).wait()


_build_table = pl.kernel(
    _build_table_body,
    out_type=jax.ShapeDtypeStruct((NUM_EMB_K * EMB_D,), jnp.float32),
    mesh=plsc.VectorSubcoreMesh(core_axis_name="c", subcore_axis_name="s", num_cores=NC, num_subcores=NS),
    scratch_types=A_SCRATCH,
    compiler_params=pltpu.CompilerParams(use_tc_tiling_on_sc=False),
)

_bag_pool = pl.kernel(
    _bag_pool_body,
    out_type=jax.ShapeDtypeStruct((BATCH_K, EMB_D), jnp.float32),
    mesh=plsc.VectorSubcoreMesh(core_axis_name="c", subcore_axis_name="s", num_cores=NC, num_subcores=NS),
    scratch_types=B_SCRATCH,
    compiler_params=pltpu.CompilerParams(use_tc_tiling_on_sc=False),
)


def kernel(x, hashed_weight, idx0, idx1):
    xflat = x.reshape(-1)
    i0 = idx0.reshape(-1)
    i1 = idx1.reshape(-1)
    tflat = _build_table(hashed_weight, i0, i1)
    table = tflat.reshape(NUM_EMB_K, EMB_D)
    return _bag_pool(table, xflat)


# speedup vs baseline: 1.0606x; 1.0182x over previous
import functools

import jax
import jax.numpy as jnp
from jax import lax
from jax.experimental import pallas as pl
from jax.experimental.pallas import tpu as pltpu
from jax.experimental.pallas import tpu_sc as plsc

NUM_EMB_K = 100000
EMB_D = 64
HASHED_K = 64000 * 10
BATCH_K = 4096
BAG_K = 50

NC = 2
NS = 16
NW = NC * NS

A_ROWS_PER_TILE = NUM_EMB_K // NW
A_CHUNK_ROWS = 125
A_CHUNKS = A_ROWS_PER_TILE // A_CHUNK_ROWS
A_CHUNK_ELEMS = A_CHUNK_ROWS * EMB_D
A_HBM_PART = 1200
A_SP_PART = A_CHUNK_ELEMS - A_HBM_PART

B_BAGS_PER_TILE = BATCH_K // NW
B_GROUP = 16
B_GROUPS = B_BAGS_PER_TILE // B_GROUP
B_GROUP_IDS = B_GROUP * BAG_K


def _wid():
    return lax.axis_index("s") * NC + lax.axis_index("c")


HW_SHARD = HASHED_K // NS
CH = A_CHUNK_ELEMS

A_SCRATCH = [
    pltpu.VMEM((2 * CH,), jnp.int32),
    pltpu.VMEM((2 * CH,), jnp.int32),
    pltpu.VMEM((2 * CH,), jnp.float32),
    pltpu.VMEM((2 * CH,), jnp.float32),
    pltpu.VMEM((2 * CH,), jnp.float32),
    pltpu.VMEM_SHARED((HASHED_K,), jnp.float32),
    pltpu.SemaphoreType.DMA,
    pltpu.SemaphoreType.DMA,
    pltpu.SemaphoreType.DMA,
    pltpu.SemaphoreType.DMA,
    pltpu.SemaphoreType.DMA,
    pltpu.SemaphoreType.DMA,
    pltpu.SemaphoreType.DMA,
    pltpu.SemaphoreType.DMA,
]


def _build_table_body(
    hw, idx0f, idx1f, tflat,
    i0d, i1d, v0d, v1d, td, hw_sh,
    gh0, gh1, gs0, gs1, ix0, ix1, wr0, wr1,
):
    sid = lax.axis_index("s")
    sem_gh = (gh0, gh1)
    sem_gs = (gs0, gs1)
    sem_ix = (ix0, ix1)
    sem_wr = (wr0, wr1)

    def stage(k, _):
        off = sid * HW_SHARD + k * CH
        pltpu.sync_copy(hw.at[pl.ds(off, CH)], v0d.at[pl.ds(0, CH)])
        pltpu.sync_copy(v0d.at[pl.ds(0, CH)], hw_sh.at[pl.ds(off, CH)])
        return 0

    lax.fori_loop(0, HW_SHARD // CH, stage, 0)
    plsc.subcore_barrier()

    base = _wid() * A_ROWS_PER_TILE * EMB_D

    def d_idx(cj, b):
        off = base + cj * CH
        return (
            pltpu.make_async_copy(
                idx0f.at[pl.ds(off, CH)], i0d.at[pl.ds(b * CH, CH)], sem_ix[b]
            ),
            pltpu.make_async_copy(
                idx1f.at[pl.ds(off, CH)], i1d.at[pl.ds(b * CH, CH)], sem_ix[b]
            ),
        )

    def d_gather(b):
        o = b * CH
        hbm_part = (
            (
                pltpu.make_async_copy(
                    hw.at[i0d.at[pl.ds(o, A_HBM_PART)]],
                    v0d.at[pl.ds(o, A_HBM_PART)],
                    sem_gh[b],
                ),
                pltpu.make_async_copy(
                    hw.at[i1d.at[pl.ds(o, A_HBM_PART)]],
                    v1d.at[pl.ds(o, A_HBM_PART)],
                    sem_gh[b],
                ),
            )
            if A_HBM_PART
            else ()
        )
        return hbm_part + (
            pltpu.make_async_copy(
                hw_sh.at[i0d.at[pl.ds(o + A_HBM_PART, A_SP_PART)]],
                v0d.at[pl.ds(o + A_HBM_PART, A_SP_PART)],
                sem_gs[b],
            ),
            pltpu.make_async_copy(
                hw_sh.at[i1d.at[pl.ds(o + A_HBM_PART, A_SP_PART)]],
                v1d.at[pl.ds(o + A_HBM_PART, A_SP_PART)],
                sem_gs[b],
            ),
        )

    def d_wr(cj, b):
        return pltpu.make_async_copy(
            td.at[pl.ds(b * CH, CH)],
            tflat.at[pl.ds(base + cj * CH, CH)],
            sem_wr[b],
        )

    for d in d_idx(0, 0):
        d.start()
    for d in d_idx(0, 0):
        d.wait()
    for d in d_gather(0):
        d.start()
    for d in d_idx(1, 1):
        d.start()

    def half_step(h, _):
        for b in range(2):
            ci = 2 * h + b
            nb = 1 - b

            @pl.when(ci < A_CHUNKS)
            def _():
                for d in d_gather(b):
                    d.wait()

                @pl.when(ci + 1 < A_CHUNKS)
                def _():
                    for d in d_idx(ci + 1, nb):
                        d.wait()
                    for d in d_gather(nb):
                        d.start()

                @pl.when(ci + 2 < A_CHUNKS)
                def _():
                    for d in d_idx(ci + 2, b):
                        d.start()

                @pl.when(ci >= 2)
                def _():
                    d_wr(ci - 2, b).wait()

                def add_vec(k, _):
                    o = b * CH + k * EMB_D
                    for u in range(4):
                        s = pl.ds(o + u * 16, 16)
                        td[s] = v0d[s] + v1d[s]
                    return 0

                lax.fori_loop(0, A_CHUNK_ROWS, add_vec, 0)
                d_wr(ci, b).start()

        return 0

    lax.fori_loop(0, (A_CHUNKS + 1) // 2, half_step, 0)
    d_wr(A_CHUNKS - 2, (A_CHUNKS - 2) % 2).wait()
    d_wr(A_CHUNKS - 1, (A_CHUNKS - 1) % 2).wait()


B_SCRATCH = [
    pltpu.VMEM((2 * B_GROUP_IDS,), jnp.int32),
    pltpu.VMEM((2 * B_GROUP_IDS, EMB_D), jnp.float32),
    pltpu.VMEM((2 * B_GROUP, EMB_D), jnp.float32),
    pltpu.SemaphoreType.DMA,
    pltpu.SemaphoreType.DMA,
    pltpu.SemaphoreType.DMA,
    pltpu.SemaphoreType.DMA,
    pltpu.SemaphoreType.DMA,
    pltpu.SemaphoreType.DMA,
]


def _bag_pool_body(
    table, xflat, out, xg_v, rows_v, out_v, gr0, gr1, ix0, ix1, wr0, wr1
):
    tile_bag0 = _wid() * B_BAGS_PER_TILE
    sem_gr = (gr0, gr1)
    sem_ix = (ix0, ix1)
    sem_wr = (wr0, wr1)

    def d_ids(ci, b):
        bag0 = tile_bag0 + ci * B_GROUP
        return pltpu.make_async_copy(
            xflat.at[pl.ds(bag0 * BAG_K, B_GROUP_IDS)],
            xg_v.at[pl.ds(b * B_GROUP_IDS, B_GROUP_IDS)],
            sem_ix[b],
        )

    def d_rows(b):
        return pltpu.make_async_copy(
            table.at[xg_v.at[pl.ds(b * B_GROUP_IDS, B_GROUP_IDS)]],
            rows_v.at[pl.ds(b * B_GROUP_IDS, B_GROUP_IDS)],
            sem_gr[b],
        )

    def d_out(ci, b):
        bag0 = tile_bag0 + ci * B_GROUP
        return pltpu.make_async_copy(
            out_v.at[pl.ds(b * B_GROUP, B_GROUP)],
            out.at[pl.ds(bag0, B_GROUP)],
            sem_wr[b],
        )

    d_ids(0, 0).start()
    d_ids(0, 0).wait()
    d_rows(0).start()
    d_ids(1, 1).start()

    def half_step(h, _):
        for b in range(2):
            ci = 2 * h + b
            nb = 1 - b

            d_rows(b).wait()

            @pl.when(ci + 1 < B_GROUPS)
            def _():
                d_ids(ci + 1, nb).wait()
                d_rows(nb).start()

            @pl.when(ci + 2 < B_GROUPS)
            def _():
                d_ids(ci + 2, b).start()

            @pl.when(ci >= 2)
            def _():
                d_out(ci - 2, b).wait()

            def bag(g, _):
                def lstep(l, acc):
                    r = b * B_GROUP_IDS + g * BAG_K + l
                    return tuple(
                        acc[c] + rows_v[r, pl.ds(c * 16, 16)] for c in range(4)
                    )

                z = jnp.zeros((16,), jnp.float32)
                acc = lax.fori_loop(0, BAG_K, lstep, (z, z, z, z))
                for c in range(4):
                    out_v[b * B_GROUP + g, pl.ds(c * 16, 16)] = acc[c]
                return 0

            lax.fori_loop(0, B_GROUP, bag, 0)
            d_out(ci, b).start()

        return 0

    lax.fori_loop(0, B_GROUPS // 2, half_step, 0)
    d_out(B_GROUPS - 2, 0).wait()
    d_out(B_GROUPS - 1, 1).wait()


_build_table = pl.kernel(
    _build_table_body,
    out_type=jax.ShapeDtypeStruct((NUM_EMB_K * EMB_D,), jnp.float32),
    mesh=plsc.VectorSubcoreMesh(core_axis_name="c", subcore_axis_name="s", num_cores=NC, num_subcores=NS),
    scratch_types=A_SCRATCH,
    compiler_params=pltpu.CompilerParams(use_tc_tiling_on_sc=False),
)

_bag_pool = pl.kernel(
    _bag_pool_body,
    out_type=jax.ShapeDtypeStruct((BATCH_K, EMB_D), jnp.float32),
    mesh=plsc.VectorSubcoreMesh(core_axis_name="c", subcore_axis_name="s", num_cores=NC, num_subcores=NS),
    scratch_types=B_SCRATCH,
    compiler_params=pltpu.CompilerParams(use_tc_tiling_on_sc=False),
)


def kernel(x, hashed_weight, idx0, idx1):
    xflat = x.reshape(-1)
    i0 = idx0.reshape(-1)
    i1 = idx1.reshape(-1)
    tflat = _build_table(hashed_weight, i0, i1)
    table = tflat.reshape(NUM_EMB_K, EMB_D)
    return _bag_pool(table, xflat)
